# Initial kernel scaffold; baseline (speedup 1.0000x reference)
import functools

import jax
import jax.numpy as jnp
from jax import lax
from jax.experimental import pallas as pl
from jax.experimental.pallas import tpu as pltpu
from jax.experimental.pallas import tpu_sc as plsc

NN = 10000
EE = 320000
DD = 128
HALF = DD // 2
NC = 2
NS = 16
PS = 640
NP = NS * PS
EROWS = 2528
ERS = EROWS // NS
DUMMY = NN
NBLK = 400
NGRID = NN // NBLK


def _mlp(v, w1, b1, w2, b2, w3, b3):
    h = jnp.maximum(jnp.dot(v, w1, preferred_element_type=jnp.float32) + b1, 0.0)
    h = jnp.maximum(jnp.dot(h, w2, preferred_element_type=jnp.float32) + b2, 0.0)
    return jnp.dot(h, w3, preferred_element_type=jnp.float32) + b3


def _ka_body(x_ref, w1, b1, w2, b2, w3, b3, out_ref):
    y = _mlp(x_ref[...], w1[...], b1[...], w2[...], b2[...], w3[...], b3[...])
    out_ref[0] = y[:, :HALF]
    out_ref[1] = y[:, HALF:]


def _run_ka(x, w1, b1, w2, b2, w3, b3):
    return pl.pallas_call(
        _ka_body,
        grid=(NGRID,),
        in_specs=[
            pl.BlockSpec((NBLK, DD), lambda i: (i, 0)),
            pl.BlockSpec(w1.shape, lambda i: (0, 0)),
            pl.BlockSpec(b1.shape, lambda i: (0, 0)),
            pl.BlockSpec(w2.shape, lambda i: (0, 0)),
            pl.BlockSpec(b2.shape, lambda i: (0, 0)),
            pl.BlockSpec(w3.shape, lambda i: (0, 0)),
            pl.BlockSpec(b3.shape, lambda i: (0, 0)),
        ],
        out_specs=pl.BlockSpec((NC, NBLK, HALF), lambda i: (0, i, 0)),
        out_shape=jax.ShapeDtypeStruct((NC, NN, HALF), jnp.float32),
    )(x, w1, b1, w2, b2, w3, b3)


def _make_sc_round(first):
    mesh = plsc.VectorSubcoreMesh(
        core_axis_name="c", subcore_axis_name="s", num_cores=NC, num_subcores=NS
    )
    out_type = [jax.ShapeDtypeStruct((NC * NP, HALF), jnp.float32)]
    if first:
        out_type.append(jax.ShapeDtypeStruct((NC, NP), jnp.float32))
    scratch = [
        pltpu.VMEM_SHARED((NP, HALF), jnp.float32),
        pltpu.VMEM((ERS, 128), jnp.int32),
        pltpu.VMEM((ERS, 128), jnp.int32),
        pltpu.VMEM((128, HALF), jnp.float32),
        pltpu.VMEM((128, HALF), jnp.float32),
        pltpu.VMEM((PS,), jnp.float32),
        pltpu.VMEM((PS,), jnp.float32),
    ]
    if first:
        scratch.append(pltpu.VMEM_SHARED((NP,), jnp.float32))
        scratch.append(pltpu.VMEM((128,), jnp.float32))

    def body(*refs):
        if first:
            (zin, src2, dstm, zeros2d, zeros1d, ones1d,
             zout, cntout,
             acc_sh, src_v, dst_v, rows_v, abuf, cnt_v, inv_v,
             cnt_sh, ones_v) = refs
        else:
            (zin, src2, dstm, zeros2d, cntin,
             zout,
             acc_sh, src_v, dst_v, rows_v, abuf, cnt_v, inv_v) = refs
        c = lax.axis_index("c")
        s = lax.axis_index("s")
        base = s * PS

        pltpu.sync_copy(zeros2d, acc_sh.at[pl.ds(base, PS)])
        if first:
            pltpu.sync_copy(zeros1d, cnt_sh.at[pl.ds(base, PS)])
            pltpu.sync_copy(ones1d, ones_v)

        pltpu.sync_copy(src2.at[c, pl.ds(s * ERS, ERS)], src_v)
        pltpu.sync_copy(dstm.at[pl.ds(s * ERS, ERS)], dst_v)

        plsc.subcore_barrier()

        def sbody(r, carry):
            pltpu.sync_copy(zin.at[src_v.at[r]], rows_v)
            pltpu.sync_copy(rows_v, acc_sh.at[dst_v.at[r]], add=True)
            if first:
                pltpu.sync_copy(ones_v, cnt_sh.at[dst_v.at[r]], add=True)
            return carry

        lax.fori_loop(0, ERS, sbody, 0)

        plsc.subcore_barrier()

        if first:
            pltpu.sync_copy(cnt_sh.at[pl.ds(base, PS)], cnt_v)
            pltpu.sync_copy(cnt_sh.at[pl.ds(base, PS)], cntout.at[c, pl.ds(base, PS)])
        else:
            pltpu.sync_copy(cntin.at[c, pl.ds(base, PS)], cnt_v)

        def ibody(k, carry):
            cv = cnt_v[pl.ds(k * 16, 16)]
            inv_v[pl.ds(k * 16, 16)] = 1.0 / jnp.maximum(cv, 1.0)
            return carry

        lax.fori_loop(0, PS // 16, ibody, 0)

        for off in range(0, PS, 128):
            pltpu.sync_copy(acc_sh.at[pl.ds(base + off, 128)], abuf)

            def cbody(r, carry):
                iv = jnp.full((16,), inv_v[off + r], jnp.float32)
                for j in range(HALF // 16):
                    abuf[r, pl.ds(j * 16, 16)] = abuf[r, pl.ds(j * 16, 16)] * iv
                return carry

            lax.fori_loop(0, 128, cbody, 0)
            pltpu.sync_copy(abuf, zout.at[pl.ds(c * NP + base + off, 128)])

    return pl.kernel(body, out_type=tuple(out_type), mesh=mesh,
                     scratch_types=tuple(scratch))


_sc_first = _make_sc_round(True)
_sc_rest = _make_sc_round(False)


def _kb_body(*refs):
    halves = refs[:16]
    w1, b1, w2, b2, w3, b3, wf1, bf1, wf2, bf2, out_ref = refs[16:]
    acc = jnp.zeros((NBLK, DD), jnp.float32)
    for k in range(8):
        v = jnp.concatenate([halves[2 * k][0], halves[2 * k + 1][0]], axis=1)
        fld = _mlp(v, w1[...], b1[...], w2[...], b2[...], w3[...], b3[...])
        acc += jnp.dot(fld, wf1[pl.ds(k * DD, DD), :],
                       preferred_element_type=jnp.float32)
    acc += bf1[...]
    logits = jnp.dot(acc, wf2[...], preferred_element_type=jnp.float32) + bf2[...]
    m = jnp.max(logits, axis=-1, keepdims=True)
    e = jnp.exp(logits - m)
    out_ref[...] = e / jnp.sum(e, axis=-1, keepdims=True)


def _run_kb(feats, w1, b1, w2, b2, w3, b3, wf1, bf1, wf2, bf2):
    in_specs = []
    ops = []
    for f in feats:
        in_specs.append(pl.BlockSpec((1, NBLK, HALF), lambda i: (0, i, 0)))
        in_specs.append(pl.BlockSpec((1, NBLK, HALF), lambda i: (1, i, 0)))
        ops.extend([f, f])
    for w in (w1, b1, w2, b2, w3, b3, wf1, bf1, wf2, bf2):
        in_specs.append(
            pl.BlockSpec(w.shape, functools.partial(lambda nd, i: (0,) * nd, w.ndim)))
        ops.append(w)
    return pl.pallas_call(
        _kb_body,
        grid=(NGRID,),
        in_specs=in_specs,
        out_specs=pl.BlockSpec((NBLK, 8), lambda i: (i, 0)),
        out_shape=jax.ShapeDtypeStruct((NN, 8), jnp.float32),
    )(*ops)


def kernel(x, edge_index, edge_attr, W1, b1, W2, b2, W3, b3, Wf1, bf1, Wf2, bf2):
    del edge_attr
    src = edge_index[0]
    dst = edge_index[1]
    pad = EROWS * 128 - EE
    srcp = jnp.concatenate([src, jnp.zeros((pad,), jnp.int32)]).reshape(EROWS, 128)
    dstp = jnp.concatenate([dst, jnp.full((pad,), DUMMY, jnp.int32)]).reshape(EROWS, 128)
    src2 = jnp.stack([srcp, srcp + NP])
    zeros2d = jnp.zeros((PS, HALF), jnp.float32)
    zeros1d = jnp.zeros((PS,), jnp.float32)
    ones1d = jnp.ones((128,), jnp.float32)

    b1r = b1.reshape(1, -1)
    b2r = b2.reshape(1, -1)
    b3r = b3.reshape(1, -1)
    bf1r = bf1.reshape(1, -1)
    bf2r = bf2.reshape(1, -1)

    y2 = _run_ka(x, W1, b1r, W2, b2r, W3, b3r)
    ypad = jnp.concatenate(
        [y2, jnp.zeros((NC, NP - NN, HALF), jnp.float32)], axis=1)
    yflat = ypad.reshape(NC * NP, HALF)

    z, cnt = _sc_first(yflat, src2, dstp, zeros2d, zeros1d, ones1d)
    zs = [z]
    for _ in range(6):
        z = _sc_rest(z, src2, dstp, zeros2d, cnt)
        zs.append(z)

    feats = [yflat.reshape(NC, NP, HALF)]
    feats += [zz.reshape(NC, NP, HALF) for zz in zs]
    return _run_kb(feats, W1, b1r, W2, b2r, W3, b3r, Wf1, bf1r, Wf2, bf2r)


# speedup vs baseline: 4.0201x; 4.0201x over previous
import functools

import jax
import jax.numpy as jnp
from jax import lax
from jax.experimental import pallas as pl
from jax.experimental.pallas import tpu as pltpu
from jax.experimental.pallas import tpu_sc as plsc

NN = 10000
EE = 320000
DD = 128
HALF = DD // 2
NC = 2
NS = 16
PS = 640
NP = NS * PS
EROWS = 2560
ERS = EROWS // NS
CHK = 32
GB = 4
DUMMY = NN
NBLK = 400
NGRID = NN // NBLK


def _mlp(v, w1, b1, w2, b2, w3, b3):
    h = jnp.maximum(jnp.dot(v, w1, preferred_element_type=jnp.float32) + b1, 0.0)
    h = jnp.maximum(jnp.dot(h, w2, preferred_element_type=jnp.float32) + b2, 0.0)
    return jnp.dot(h, w3, preferred_element_type=jnp.float32) + b3


def _ka_body(x_ref, w1, b1, w2, b2, w3, b3, out_ref):
    y = _mlp(x_ref[...], w1[...], b1[...], w2[...], b2[...], w3[...], b3[...])
    out_ref[0] = y[:, :HALF]
    out_ref[1] = y[:, HALF:]


def _run_ka(x, w1, b1, w2, b2, w3, b3):
    return pl.pallas_call(
        _ka_body,
        grid=(NGRID,),
        in_specs=[
            pl.BlockSpec((NBLK, DD), lambda i: (i, 0)),
            pl.BlockSpec(w1.shape, lambda i: (0, 0)),
            pl.BlockSpec(b1.shape, lambda i: (0, 0)),
            pl.BlockSpec(w2.shape, lambda i: (0, 0)),
            pl.BlockSpec(b2.shape, lambda i: (0, 0)),
            pl.BlockSpec(w3.shape, lambda i: (0, 0)),
            pl.BlockSpec(b3.shape, lambda i: (0, 0)),
        ],
        out_specs=pl.BlockSpec((NC, NBLK, HALF), lambda i: (0, i, 0)),
        out_shape=jax.ShapeDtypeStruct((NC, NN, HALF), jnp.float32),
    )(x, w1, b1, w2, b2, w3, b3)


@functools.cache
def _make_sc_round(first):
    mesh = plsc.VectorSubcoreMesh(
        core_axis_name="c", subcore_axis_name="s", num_cores=NC, num_subcores=NS
    )
    out_type = [jax.ShapeDtypeStruct((NC * NP, HALF), jnp.float32)]
    if first:
        out_type.append(jax.ShapeDtypeStruct((NC, NP), jnp.float32))
    scratch = [
        pltpu.VMEM_SHARED((NP, HALF), jnp.float32),
        pltpu.VMEM((ERS, 128), jnp.int32),
        pltpu.VMEM((ERS, 128), jnp.int32),
        pltpu.VMEM((2, 128, HALF), jnp.float32),
        pltpu.VMEM((2, 128, HALF), jnp.float32),
        pltpu.VMEM((128, HALF), jnp.float32),
        pltpu.VMEM((PS,), jnp.float32),
        pltpu.VMEM((PS,), jnp.float32),
        pltpu.SemaphoreType.DMA,
        pltpu.SemaphoreType.DMA,
    ]
    if first:
        scratch.append(pltpu.VMEM_SHARED((NP,), jnp.float32))
        scratch.append(pltpu.VMEM((128,), jnp.float32))

    def body(*refs):
        if first:
            (zin, src2, dstm, zeros2d, zeros1d, ones1d,
             zout, cntout,
             acc_sh, src_v, dst_v, rows_a, rows_b, abuf, cnt_v, inv_v,
             gsem_a, gsem_b,
             cnt_sh, ones_v) = refs
        else:
            (zin, src2, dstm, zeros2d, cntin,
             zout,
             acc_sh, src_v, dst_v, rows_a, rows_b, abuf, cnt_v, inv_v,
             gsem_a, gsem_b) = refs
        c = lax.axis_index("c")
        s = lax.axis_index("s")
        base = s * PS

        pltpu.sync_copy(zeros2d, acc_sh.at[pl.ds(base, PS)])
        if first:
            pltpu.sync_copy(zeros1d, cnt_sh.at[pl.ds(base, PS)])
            pltpu.sync_copy(ones1d, ones_v)

        pltpu.sync_copy(src2.at[c, pl.ds(s * ERS, ERS)], src_v)
        pltpu.sync_copy(dstm.at[pl.ds(s * ERS, ERS)], dst_v)

        plsc.subcore_barrier()

        def gather(bufs, lo, sem):
            for j in range(2):
                pltpu.async_copy(zin.at[src_v.at[lo + j]], bufs.at[j], sem)

        def gather_wait(bufs, lo, sem):
            for j in range(2):
                pltpu.make_async_copy(
                    zin.at[src_v.at[lo + j]], bufs.at[j], sem).wait()

        def scatter(bufs, lo):
            for j in range(2):
                pltpu.sync_copy(bufs.at[j], acc_sh.at[dst_v.at[lo + j]],
                                add=True)
                if first:
                    pltpu.sync_copy(ones_v, cnt_sh.at[dst_v.at[lo + j]],
                                    add=True)

        gather(rows_a, 0, gsem_a)

        def sbody(k, carry):
            ra = 4 * k
            rc = jnp.minimum(ra + 4, ERS - 2)
            gather_wait(rows_a, ra, gsem_a)
            gather(rows_b, ra + 2, gsem_b)
            scatter(rows_a, ra)
            gather_wait(rows_b, ra + 2, gsem_b)
            gather(rows_a, rc, gsem_a)
            scatter(rows_b, ra + 2)
            return carry

        lax.fori_loop(0, ERS // 4, sbody, 0)
        gather_wait(rows_a, ERS - 2, gsem_a)

        plsc.subcore_barrier()

        if first:
            pltpu.sync_copy(cnt_sh.at[pl.ds(base, PS)], cnt_v)
            pltpu.sync_copy(cnt_sh.at[pl.ds(base, PS)], cntout.at[c, pl.ds(base, PS)])
        else:
            pltpu.sync_copy(cntin.at[c, pl.ds(base, PS)], cnt_v)

        def ibody(k, carry):
            cv = cnt_v[pl.ds(k * 16, 16)]
            inv_v[pl.ds(k * 16, 16)] = 1.0 / jnp.maximum(cv, 1.0)
            return carry

        lax.fori_loop(0, PS // 16, ibody, 0)

        for off in range(0, PS, 128):
            pltpu.sync_copy(acc_sh.at[pl.ds(base + off, 128)], abuf)

            def cbody(g, carry):
                iv16 = inv_v[pl.ds(off + g * 16, 16)]
                for r16 in range(16):
                    ivr = jnp.broadcast_to(iv16[r16], (16,))
                    row = g * 16 + r16
                    for j in range(HALF // 16):
                        abuf[row, pl.ds(j * 16, 16)] = (
                            abuf[row, pl.ds(j * 16, 16)] * ivr)
                return carry

            lax.fori_loop(0, 8, cbody, 0)
            pltpu.sync_copy(abuf, zout.at[pl.ds(c * NP + base + off, 128)])

    return pl.kernel(body, out_type=tuple(out_type), mesh=mesh,
                     scratch_types=tuple(scratch),
                     compiler_params=pltpu.CompilerParams(
                         use_tc_tiling_on_sc=False))


def _kb_body(*refs):
    halves = refs[:16]
    w1, b1, w2, b2, w3, b3, wf1, bf1, wf2, bf2, out_ref = refs[16:]
    acc = jnp.zeros((NBLK, DD), jnp.float32)
    for k in range(8):
        v = jnp.concatenate([halves[2 * k][0], halves[2 * k + 1][0]], axis=1)
        fld = _mlp(v, w1[...], b1[...], w2[...], b2[...], w3[...], b3[...])
        acc += jnp.dot(fld, wf1[pl.ds(k * DD, DD), :],
                       preferred_element_type=jnp.float32)
    acc += bf1[...]
    logits = jnp.dot(acc, wf2[...], preferred_element_type=jnp.float32) + bf2[...]
    m = jnp.max(logits, axis=-1, keepdims=True)
    e = jnp.exp(logits - m)
    out_ref[...] = e / jnp.sum(e, axis=-1, keepdims=True)


def _run_kb(feats, w1, b1, w2, b2, w3, b3, wf1, bf1, wf2, bf2):
    in_specs = []
    ops = []
    for f in feats:
        in_specs.append(pl.BlockSpec((1, NBLK, HALF), lambda i: (0, i, 0)))
        in_specs.append(pl.BlockSpec((1, NBLK, HALF), lambda i: (1, i, 0)))
        ops.extend([f, f])
    for w in (w1, b1, w2, b2, w3, b3, wf1, bf1, wf2, bf2):
        in_specs.append(
            pl.BlockSpec(w.shape, functools.partial(lambda nd, i: (0,) * nd, w.ndim)))
        ops.append(w)
    return pl.pallas_call(
        _kb_body,
        grid=(NGRID,),
        in_specs=in_specs,
        out_specs=pl.BlockSpec((NBLK, 8), lambda i: (i, 0)),
        out_shape=jax.ShapeDtypeStruct((NN, 8), jnp.float32),
    )(*ops)


def kernel(x, edge_index, edge_attr, W1, b1, W2, b2, W3, b3, Wf1, bf1, Wf2, bf2):
    del edge_attr
    src = edge_index[0]
    dst = edge_index[1]
    pad = EROWS * 128 - EE
    srcp = jnp.concatenate([src, jnp.zeros((pad,), jnp.int32)]).reshape(EROWS, 128)
    dstp = jnp.concatenate([dst, jnp.full((pad,), DUMMY, jnp.int32)]).reshape(EROWS, 128)
    src2 = jnp.stack([srcp, srcp + NP])
    zeros2d = jnp.zeros((PS, HALF), jnp.float32)
    zeros1d = jnp.zeros((PS,), jnp.float32)
    ones1d = jnp.ones((128,), jnp.float32)

    b1r = b1.reshape(1, -1)
    b2r = b2.reshape(1, -1)
    b3r = b3.reshape(1, -1)
    bf1r = bf1.reshape(1, -1)
    bf2r = bf2.reshape(1, -1)

    y2 = _run_ka(x, W1, b1r, W2, b2r, W3, b3r)
    ypad = jnp.concatenate(
        [y2, jnp.zeros((NC, NP - NN, HALF), jnp.float32)], axis=1)
    yflat = ypad.reshape(NC * NP, HALF)

    z, cnt = _make_sc_round(True)(yflat, src2, dstp, zeros2d, zeros1d, ones1d)
    zs = [z]
    for _ in range(6):
        (z,) = _make_sc_round(False)(z, src2, dstp, zeros2d, cnt)
        zs.append(z)

    feats = [yflat.reshape(NC, NP, HALF)]
    feats += [zz.reshape(NC, NP, HALF) for zz in zs]
    return _run_kb(feats, W1, b1r, W2, b2r, W3, b3r, Wf1, bf1r, Wf2, bf2r)
